# baseline (speedup 1.0000x reference)
import functools
import math

import jax
import jax.numpy as jnp
from jax import lax
from jax.experimental import pallas as pl
from jax.experimental.pallas import tpu as pltpu
from jax.experimental.pallas import tpu_sc as plsc

D_MODEL = 64
SCALE = math.sqrt(D_MODEL)
NC = 2
NS = 16
NW = NC * NS
CH = 512


@functools.lru_cache(maxsize=None)
def _make(B):
    b_per_w = B // NW
    nchunk = b_per_w // CH
    mesh = plsc.VectorSubcoreMesh(core_axis_name="c", subcore_axis_name="s")

    @functools.partial(
        pl.kernel,
        mesh=mesh,
        compiler_params=pltpu.CompilerParams(use_tc_tiling_on_sc=False),
        out_type=jax.ShapeDtypeStruct((B, D_MODEL), jnp.float32),
        scratch_types=[
            pltpu.VMEM((CH,), jnp.int32),
            pltpu.VMEM((CH, D_MODEL), jnp.float32),
            pltpu.SemaphoreType.DMA,
        ],
    )
    def emb(idx_hbm, table_hbm, out_hbm, idx_v, rows_v, sem):
        wid = lax.axis_index("s") * NC + lax.axis_index("c")
        base = wid * b_per_w

        def chunk(g, carry):
            off = base + g * CH
            pltpu.sync_copy(idx_hbm.at[pl.ds(off, CH)], idx_v)
            pltpu.async_copy(table_hbm.at[idx_v], rows_v, sem).wait()

            def scale_rows(r, c):
                for rr in range(8):
                    for j in range(D_MODEL // 16):
                        sl = (r * 8 + rr, pl.ds(j * 16, 16))
                        rows_v[sl] = rows_v[sl] * SCALE
                return c

            lax.fori_loop(0, CH // 8, scale_rows, 0)
            pltpu.sync_copy(rows_v, out_hbm.at[pl.ds(off, CH)])
            return carry

        lax.fori_loop(0, nchunk, chunk, 0)

    return emb


def kernel(indices, table):
    bsz, hist = indices.shape
    flat = indices.reshape(-1).astype(jnp.int32)
    out = _make(flat.shape[0])(flat, table)
    return out.reshape(bsz, hist, D_MODEL)


# speedup vs baseline: 1.0495x; 1.0495x over previous
import functools
import math

import jax
import jax.numpy as jnp
from jax import lax
from jax.experimental import pallas as pl
from jax.experimental.pallas import tpu as pltpu
from jax.experimental.pallas import tpu_sc as plsc

D_MODEL = 64
SCALE = math.sqrt(D_MODEL)
NC = 2
NS = 16
NW = NC * NS
CH = 320
NBUF = 4
RU = 8


@functools.lru_cache(maxsize=None)
def _make(B):
    b_per_w = B // NW
    nchunk = b_per_w // CH
    nouter = nchunk // NBUF
    mesh = plsc.VectorSubcoreMesh(core_axis_name="c", subcore_axis_name="s")

    @functools.partial(
        pl.kernel,
        mesh=mesh,
        compiler_params=pltpu.CompilerParams(use_tc_tiling_on_sc=False),
        out_type=jax.ShapeDtypeStruct((B, D_MODEL), jnp.float32),
        scratch_types=[
            pltpu.VMEM((nchunk, CH), jnp.int32),
            pltpu.VMEM((NBUF, CH, D_MODEL), jnp.float32),
        ]
        + [pltpu.SemaphoreType.DMA] * (2 * NBUF),
    )
    def emb(idx_hbm, table_hbm, out_hbm, idx_v, rows_v, *sems):
        gsem = sems[:NBUF]
        osem = sems[NBUF:]
        wid = lax.axis_index("s") * NC + lax.axis_index("c")
        row0 = wid * nchunk
        base = row0 * CH

        pltpu.sync_copy(idx_hbm.at[pl.ds(row0, nchunk)], idx_v)

        for b in range(NBUF):
            pltpu.async_copy(table_hbm.at[idx_v.at[b]], rows_v.at[b], gsem[b])

        def chunk_body(t, b, issue_next):
            g = t * NBUF + b
            off = base + g * CH
            rv = rows_v.at[b]
            pltpu.make_async_copy(table_hbm.at[idx_v.at[b]], rv, gsem[b]).wait()

            def scale_rows(r, c):
                for rr in range(RU):
                    row = r * RU + rr
                    for j in range(D_MODEL // 16):
                        sl = (row, pl.ds(j * 16, 16))
                        rv[sl] = rv[sl] * SCALE
                return c

            lax.fori_loop(0, CH // RU, scale_rows, 0)
            out_slice = out_hbm.at[pl.ds(off, CH)]
            pltpu.async_copy(rv, out_slice, osem[b])
            if issue_next:
                pltpu.make_async_copy(rv, out_slice, osem[b]).wait()
                pltpu.async_copy(
                    table_hbm.at[idx_v.at[g + NBUF]], rv, gsem[b]
                )
            return 0

        def outer(t, c):
            for b in range(NBUF):
                chunk_body(t, b, True)
            return c

        lax.fori_loop(0, nouter - 1, outer, 0)
        for b in range(NBUF):
            chunk_body(nouter - 1, b, False)
        for b in range(NBUF):
            g = (nouter - 1) * NBUF + b
            off = base + g * CH
            pltpu.make_async_copy(
                rows_v.at[b], out_hbm.at[pl.ds(off, CH)], osem[b]
            ).wait()

    return emb


def kernel(indices, table):
    bsz, hist = indices.shape
    n = bsz * hist
    flat = indices.reshape(n // CH, CH).astype(jnp.int32)
    out = _make(n)(flat, table)
    return out.reshape(bsz, hist, D_MODEL)


# speedup vs baseline: 1.0633x; 1.0132x over previous
import functools
import math

import jax
import jax.numpy as jnp
from jax import lax
from jax.experimental import pallas as pl
from jax.experimental.pallas import tpu as pltpu
from jax.experimental.pallas import tpu_sc as plsc

D_MODEL = 64
SCALE = math.sqrt(D_MODEL)
NC = 2
NS = 16
NW = NC * NS
CH = 128
NBUF = 2
L = 16


@functools.lru_cache(maxsize=None)
def _make(H, BSZ):
    cols_per_w = BSZ // NW
    sub_per_h = cols_per_w // CH
    nchunk = H * sub_per_h
    mesh = plsc.VectorSubcoreMesh(core_axis_name="c", subcore_axis_name="s")

    @functools.partial(
        pl.kernel,
        mesh=mesh,
        compiler_params=pltpu.CompilerParams(use_tc_tiling_on_sc=True),
        out_type=jax.ShapeDtypeStruct((H, BSZ, D_MODEL), jnp.float32),
        scratch_types=[
            pltpu.VMEM((H, cols_per_w), jnp.int32),
            pltpu.VMEM((NBUF, CH), jnp.int32),
            pltpu.VMEM((NBUF, CH, 2 * D_MODEL), jnp.float32),
            pltpu.VMEM((NBUF, CH, D_MODEL), jnp.float32),
        ]
        + [pltpu.SemaphoreType.DMA] * (2 * NBUF),
    )
    def emb(idx_hbm, table_hbm, out_hbm, idx_v, pidx_v, rows_v, obuf_v, *sems):
        gsem = sems[:NBUF]
        osem = sems[NBUF:]
        wid = lax.axis_index("s") * NC + lax.axis_index("c")
        col0 = wid * cols_per_w

        pltpu.sync_copy(idx_hbm.at[:, pl.ds(col0, cols_per_w)], idx_v)

        def fill_pidx(c, b):
            h = c // sub_per_h
            base = (c % sub_per_h) * CH
            for k in range(CH // L):
                v = idx_v[h, pl.ds(base + k * L, L)]
                pidx_v[b, pl.ds(k * L, L)] = lax.shift_right_logical(v, 1)

        def start_gather(c, b):
            fill_pidx(c, b)
            pltpu.async_copy(table_hbm.at[pidx_v.at[b]], rows_v.at[b], gsem[b])

        def wait_gather(b):
            pltpu.make_async_copy(
                table_hbm.at[pidx_v.at[b]], rows_v.at[b], gsem[b]
            ).wait()

        def out_slice(c):
            h = c // sub_per_h
            base = col0 + (c % sub_per_h) * CH
            return out_hbm.at[h, pl.ds(base, CH)]

        def wait_write(c, b):
            pltpu.make_async_copy(obuf_v.at[b], out_slice(c), osem[b]).wait()

        def dehalve_scale(c, b):
            h = c // sub_per_h
            base = (c % sub_per_h) * CH

            def group_body(rg, carry):
                v = idx_v[h, pl.ds(base + rg * L, L)]
                offs = lax.shift_left(v & 1, 6)
                for lane in range(L):
                    off = offs[lane]
                    r = rg * L + lane
                    for j in range(D_MODEL // L):
                        src = rows_v[b, r, pl.ds(off + j * L, L)]
                        obuf_v[b, r, pl.ds(j * L, L)] = src * SCALE
                return carry

            lax.fori_loop(0, CH // L, group_body, 0)

        for b in range(NBUF):
            start_gather(b, b)

        def chunk_body(c, b, first, last):
            wait_gather(b)
            if not first:
                wait_write(c - NBUF, b)
            dehalve_scale(c, b)
            if not last:
                start_gather(c + NBUF, b)
            pltpu.async_copy(obuf_v.at[b], out_slice(c), osem[b])
            return 0

        def steady(t, carry):
            for b in range(NBUF):
                chunk_body(NBUF + t * NBUF + b, b, False, False)
            return carry

        for b in range(NBUF):
            chunk_body(b, b, True, False)
        lax.fori_loop(0, (nchunk - 2 * NBUF) // NBUF, steady, 0)
        for c in range(nchunk - NBUF, nchunk):
            chunk_body(c, c % NBUF, False, True)
        for c in range(nchunk - NBUF, nchunk):
            wait_write(c, c % NBUF)

    return emb


def kernel(indices, table):
    bsz, hist = indices.shape
    nodes, d = table.shape
    pair_table = table.reshape(nodes // 2, 2 * d)
    out_t = _make(hist, bsz)(indices.T.astype(jnp.int32), pair_table)
    return out_t.transpose(1, 0, 2)


# speedup vs baseline: 1.7528x; 1.6484x over previous
import functools
import math

import jax
import jax.numpy as jnp
from jax import lax
from jax.experimental import pallas as pl
from jax.experimental.pallas import tpu as pltpu
from jax.experimental.pallas import tpu_sc as plsc

D_MODEL = 64
SCALE = math.sqrt(D_MODEL)
NC = 2
NS = 16
NW = NC * NS
CH = 128
NBUF = 4
L = 16
RU = 8


@functools.lru_cache(maxsize=None)
def _make(H, BSZ):
    cols_per_w = BSZ // NW
    sub_per_h = cols_per_w // CH
    nchunk = H * sub_per_h
    mesh = plsc.VectorSubcoreMesh(core_axis_name="c", subcore_axis_name="s")

    @functools.partial(
        pl.kernel,
        mesh=mesh,
        compiler_params=pltpu.CompilerParams(use_tc_tiling_on_sc=True),
        out_type=jax.ShapeDtypeStruct((H, BSZ, D_MODEL), jnp.float32),
        scratch_types=[
            pltpu.VMEM((H, cols_per_w), jnp.int32),
            pltpu.VMEM((NBUF, CH, D_MODEL), jnp.float32),
        ]
        + [pltpu.SemaphoreType.DMA] * (2 * NBUF),
    )
    def emb(idx_hbm, table_hbm, out_hbm, idx_v, rows_v, *sems):
        gsem = sems[:NBUF]
        osem = sems[NBUF:]
        wid = lax.axis_index("s") * NC + lax.axis_index("c")
        col0 = wid * cols_per_w

        pltpu.sync_copy(idx_hbm.at[:, pl.ds(col0, cols_per_w)], idx_v)

        def start_gather(c, b):
            h = c // sub_per_h
            base = (c % sub_per_h) * CH

            def issue(k, carry):
                v = idx_v[h, pl.ds(base + k * L, L)]
                for lane in range(L):
                    pltpu.async_copy(
                        table_hbm.at[v[lane]],
                        rows_v.at[b, k * L + lane],
                        gsem[b],
                    )
                return carry

            lax.fori_loop(0, CH // L, issue, 0)

        def wait_gather(b):
            pltpu.make_async_copy(
                table_hbm.at[pl.ds(0, CH)], rows_v.at[b], gsem[b]
            ).wait()

        def out_slice(c):
            h = c // sub_per_h
            base = col0 + (c % sub_per_h) * CH
            return out_hbm.at[h, pl.ds(base, CH)]

        def scale_chunk(b):
            def scale_rows(r, carry):
                for rr in range(RU):
                    row = r * RU + rr
                    for j in range(D_MODEL // L):
                        sl = (row, pl.ds(j * L, L))
                        rows_v[b, *sl] = rows_v[b, *sl] * SCALE
                return carry

            lax.fori_loop(0, CH // RU, scale_rows, 0)

        for b in range(NBUF):
            start_gather(b, b)

        def chunk_body(c, b, issue_next):
            wait_gather(b)
            scale_chunk(b)
            pltpu.async_copy(rows_v.at[b], out_slice(c), osem[b])
            if issue_next:
                pltpu.make_async_copy(rows_v.at[b], out_slice(c), osem[b]).wait()
                start_gather(c + NBUF, b)
            return 0

        def steady(t, carry):
            for b in range(NBUF):
                chunk_body(t * NBUF + b, b, True)
            return carry

        lax.fori_loop(0, nchunk // NBUF - 1, steady, 0)
        for b in range(NBUF):
            chunk_body(nchunk - NBUF + b, b, False)
        for b in range(NBUF):
            c = nchunk - NBUF + b
            pltpu.make_async_copy(rows_v.at[b], out_slice(c), osem[b]).wait()

    return emb


def kernel(indices, table):
    bsz, hist = indices.shape
    out_t = _make(hist, bsz)(indices.T.astype(jnp.int32), table)
    return out_t.transpose(1, 0, 2)


# speedup vs baseline: 2.2849x; 1.3036x over previous
import functools
import math

import jax
import jax.numpy as jnp
from jax import lax
from jax.experimental import pallas as pl
from jax.experimental.pallas import tpu as pltpu
from jax.experimental.pallas import tpu_sc as plsc

D_MODEL = 64
SCALE = math.sqrt(D_MODEL)
NC = 2
NS = 16
NW = NC * NS
CH = 128
NBUF = 4
L = 16
RU = 8


@functools.lru_cache(maxsize=None)
def _make(H, BSZ):
    cols_per_w = BSZ // NW
    sub_per_h = cols_per_w // CH
    nchunk = H * sub_per_h
    mesh = plsc.VectorSubcoreMesh(core_axis_name="c", subcore_axis_name="s")

    @functools.partial(
        pl.kernel,
        mesh=mesh,
        compiler_params=pltpu.CompilerParams(use_tc_tiling_on_sc=True),
        out_type=jax.ShapeDtypeStruct((H, BSZ, D_MODEL), jnp.float32),
        scratch_types=[
            pltpu.VMEM((H, cols_per_w), jnp.int32),
            pltpu.VMEM((NBUF, CH, D_MODEL), jnp.float32),
        ]
        + [pltpu.SemaphoreType.DMA] * (2 * NBUF),
    )
    def emb(idx_hbm, table_hbm, out_hbm, idx_v, rows_v, *sems):
        gsem = sems[:NBUF]
        osem = sems[NBUF:]
        wid = lax.axis_index("s") * NC + lax.axis_index("c")
        col0 = wid * cols_per_w

        pltpu.sync_copy(idx_hbm.at[:, pl.ds(col0, cols_per_w)], idx_v)

        def start_gather(c, b):
            h = c // sub_per_h
            base = (c % sub_per_h) * CH

            def issue(k, carry):
                v = idx_v[h, pl.ds(base + k * L, L)]
                for lane in range(L):
                    pltpu.async_copy(
                        table_hbm.at[0, v[lane]],
                        rows_v.at[b, k * L + lane],
                        gsem[b],
                    )
                return carry

            lax.fori_loop(0, CH // L, issue, 0)

        def wait_gather(b):
            pltpu.make_async_copy(
                table_hbm.at[0, pl.ds(0, CH)], rows_v.at[b], gsem[b]
            ).wait()

        def out_slice(c):
            h = c // sub_per_h
            base = col0 + (c % sub_per_h) * CH
            return out_hbm.at[h, pl.ds(base, CH)]

        def scale_chunk(b):
            def scale_rows(r, carry):
                for rr in range(RU):
                    row = r * RU + rr
                    for j in range(D_MODEL // L):
                        sl = (row, pl.ds(j * L, L))
                        rows_v[b, *sl] = rows_v[b, *sl] * SCALE
                return carry

            lax.fori_loop(0, CH // RU, scale_rows, 0)

        for b in range(NBUF):
            start_gather(b, b)

        def chunk_body(c, b, issue_next):
            wait_gather(b)
            scale_chunk(b)
            pltpu.async_copy(rows_v.at[b], out_slice(c), osem[b])
            if issue_next:
                pltpu.make_async_copy(rows_v.at[b], out_slice(c), osem[b]).wait()
                start_gather(c + NBUF, b)
            return 0

        def steady(t, carry):
            for b in range(NBUF):
                chunk_body(t * NBUF + b, b, True)
            return carry

        lax.fori_loop(0, nchunk // NBUF - 1, steady, 0)
        for b in range(NBUF):
            chunk_body(nchunk - NBUF + b, b, False)
        for b in range(NBUF):
            c = nchunk - NBUF + b
            pltpu.make_async_copy(rows_v.at[b], out_slice(c), osem[b]).wait()

    return emb


def kernel(indices, table):
    bsz, hist = indices.shape
    nodes, d = table.shape
    table3 = table.reshape(1, nodes, d)
    out_t = _make(hist, bsz)(indices.T.astype(jnp.int32), table3)
    return out_t.transpose(1, 0, 2)
